# Initial kernel scaffold; baseline (speedup 1.0000x reference)
import jax
import jax.numpy as jnp
from jax import lax
from jax.experimental import pallas as pl
from jax.experimental.pallas import tpu as pltpu
from jax.experimental.pallas import tpu_sc as plsc

N = 10000
E = 320000
D = 128

NC = 2
NS = 16
NW = NC * NS
EPW = E // NW
CH = 80
NCH = EPW // CH

_ROWS_A = 624
_LAST_OFF = 15 * _ROWS_A
_ROWS_LAST = N - _LAST_OFF

_mesh = plsc.VectorSubcoreMesh(core_axis_name="c", subcore_axis_name="s")


def _sc_deg_body(dst_hbm, w_hbm, z1_hbm, out_hbm, dstv, w_v, deg_sh):
    cid = lax.axis_index("c")
    sid = lax.axis_index("s")
    wid = sid * NC + cid
    roff = sid * _ROWS_A

    @pl.when(sid < 15)
    def _():
        pltpu.sync_copy(z1_hbm.at[pl.ds(roff, _ROWS_A)],
                        deg_sh.at[pl.ds(roff, _ROWS_A)])

    @pl.when(sid == 15)
    def _():
        pltpu.sync_copy(z1_hbm.at[pl.ds(_LAST_OFF, _ROWS_LAST)],
                        deg_sh.at[pl.ds(_LAST_OFF, _ROWS_LAST)])

    plsc.subcore_barrier()

    ebase = wid * EPW

    def chunk(k, carry):
        b = ebase + k * CH
        pltpu.sync_copy(dst_hbm.at[pl.ds(b, CH)], dstv)
        pltpu.sync_copy(w_hbm.at[pl.ds(b, CH)], w_v)
        pltpu.sync_copy(w_v, deg_sh.at[dstv], add=True)
        return carry

    lax.fori_loop(0, NCH, chunk, 0)
    plsc.subcore_barrier()

    @pl.when(sid < 15)
    def _():
        pltpu.sync_copy(deg_sh.at[pl.ds(roff, _ROWS_A)],
                        out_hbm.at[cid, pl.ds(roff, _ROWS_A)])

    @pl.when(sid == 15)
    def _():
        pltpu.sync_copy(deg_sh.at[pl.ds(_LAST_OFF, _ROWS_LAST)],
                        out_hbm.at[cid, pl.ds(_LAST_OFF, _ROWS_LAST)])


_sc_deg = pl.kernel(
    _sc_deg_body,
    out_type=jax.ShapeDtypeStruct((NC, N), jnp.float32),
    mesh=_mesh,
    scratch_types=[
        pltpu.VMEM((CH,), jnp.int32),
        pltpu.VMEM((CH,), jnp.float32),
        pltpu.VMEM_SHARED((N,), jnp.float32),
    ],
)


def _sc_agg_body(g_hbm, src_hbm, dst_hbm, w_hbm, z2_hbm, out_hbm,
                 srcv, dstv, w_v, rows_v, sem, acc_sh):
    cid = lax.axis_index("c")
    sid = lax.axis_index("s")
    wid = sid * NC + cid
    roff = sid * _ROWS_A

    @pl.when(sid < 15)
    def _():
        pltpu.sync_copy(z2_hbm.at[pl.ds(roff, _ROWS_A)],
                        acc_sh.at[pl.ds(roff, _ROWS_A)])

    @pl.when(sid == 15)
    def _():
        pltpu.sync_copy(z2_hbm.at[pl.ds(_LAST_OFF, _ROWS_LAST)],
                        acc_sh.at[pl.ds(_LAST_OFF, _ROWS_LAST)])

    plsc.subcore_barrier()

    ebase = wid * EPW

    def chunk(k, carry):
        b = ebase + k * CH
        pltpu.sync_copy(src_hbm.at[pl.ds(b, CH)], srcv)
        pltpu.sync_copy(dst_hbm.at[pl.ds(b, CH)], dstv)
        pltpu.sync_copy(w_hbm.at[pl.ds(b, CH)], w_v)
        pltpu.async_copy(g_hbm.at[srcv], rows_v, sem).wait()

        def escale(e, c2):
            sp = plsc.load_gather(w_v, [jnp.full((16,), e, jnp.int32)])
            for j in range(8):
                sl = pl.ds(j * 16, 16)
                rows_v[e, sl] = rows_v[e, sl] * sp
            return c2

        lax.fori_loop(0, CH, escale, 0)
        pltpu.sync_copy(rows_v, acc_sh.at[dstv], add=True)
        return carry

    lax.fori_loop(0, NCH, chunk, 0)
    plsc.subcore_barrier()

    @pl.when(sid < 15)
    def _():
        pltpu.sync_copy(acc_sh.at[pl.ds(roff, _ROWS_A)],
                        out_hbm.at[cid, pl.ds(roff, _ROWS_A)])

    @pl.when(sid == 15)
    def _():
        pltpu.sync_copy(acc_sh.at[pl.ds(_LAST_OFF, _ROWS_LAST)],
                        out_hbm.at[cid, pl.ds(_LAST_OFF, _ROWS_LAST)])


_sc_agg = pl.kernel(
    _sc_agg_body,
    out_type=jax.ShapeDtypeStruct((NC, N, D), jnp.float32),
    mesh=_mesh,
    scratch_types=[
        pltpu.VMEM((CH,), jnp.int32),
        pltpu.VMEM((CH,), jnp.int32),
        pltpu.VMEM((CH,), jnp.float32),
        pltpu.VMEM((CH, D), jnp.float32),
        pltpu.SemaphoreType.DMA,
        pltpu.VMEM_SHARED((N, D), jnp.float32),
    ],
)


def _tc_pre_body(x_ref, w_ref, deg_ref, g_ref, dis_ref):
    h = jnp.dot(x_ref[...], w_ref[...], preferred_element_type=jnp.float32)
    deg = deg_ref[0, :, 0] + deg_ref[1, :, 0] + 1.0
    dis = lax.rsqrt(deg)
    g_ref[...] = h * dis[:, None]
    dis_ref[...] = dis[:, None]


_BR = 1000

_tc_pre = pl.pallas_call(
    _tc_pre_body,
    grid=(N // _BR,),
    in_specs=[
        pl.BlockSpec((_BR, D), lambda i: (i, 0)),
        pl.BlockSpec((D, D), lambda i: (0, 0)),
        pl.BlockSpec((NC, _BR, 1), lambda i: (0, i, 0)),
    ],
    out_specs=[
        pl.BlockSpec((_BR, D), lambda i: (i, 0)),
        pl.BlockSpec((_BR, 1), lambda i: (i, 0)),
    ],
    out_shape=[
        jax.ShapeDtypeStruct((N, D), jnp.float32),
        jax.ShapeDtypeStruct((N, 1), jnp.float32),
    ],
)


def _tc_post_body(acc_ref, g_ref, dis_ref, b_ref, o_ref):
    s = acc_ref[0] + acc_ref[1] + g_ref[...]
    o_ref[...] = jnp.maximum(s * dis_ref[...] + b_ref[...], 0.0)


_tc_post = pl.pallas_call(
    _tc_post_body,
    grid=(N // _BR,),
    in_specs=[
        pl.BlockSpec((NC, _BR, D), lambda i: (0, i, 0)),
        pl.BlockSpec((_BR, D), lambda i: (i, 0)),
        pl.BlockSpec((_BR, 1), lambda i: (i, 0)),
        pl.BlockSpec((1, D), lambda i: (0, 0)),
    ],
    out_specs=pl.BlockSpec((_BR, D), lambda i: (i, 0)),
    out_shape=jax.ShapeDtypeStruct((N, D), jnp.float32),
)


def kernel(x, edge_index, edge_weights, W, b):
    src = edge_index[0]
    dst = edge_index[1]
    z1 = jnp.zeros((N,), jnp.float32)
    z2 = jnp.zeros((N, D), jnp.float32)
    deg2 = _sc_deg(dst, edge_weights, z1)
    g, dis = _tc_pre(x, W, deg2.reshape(NC, N, 1))
    acc2 = _sc_agg(g, src, dst, edge_weights, z2)
    return _tc_post(acc2, g, dis, b.reshape(1, D))


# speedup vs baseline: 13.0682x; 13.0682x over previous
import jax
import jax.numpy as jnp
from jax import lax
from jax.experimental import pallas as pl
from jax.experimental.pallas import tpu as pltpu
from jax.experimental.pallas import tpu_sc as plsc

N = 10000
E = 320000
D = 128

NC = 2
NS = 16
NW = NC * NS
EPW = E // NW
CH = 80
NCH = EPW // CH

_STRIDE = 624
_ROWS = 640

_mesh = plsc.VectorSubcoreMesh(core_axis_name="c", subcore_axis_name="s")


def _sc_deg_body(dst_hbm, w_hbm, out_hbm, dstv, w_v, dbuf, deg_sh):
    cid = lax.axis_index("c")
    sid = lax.axis_index("s")
    wid = sid * NC + cid
    roff = sid * _STRIDE

    for t in range(_ROWS // 16):
        dbuf[pl.ds(t * 16, 16)] = jnp.zeros((16,), jnp.float32)
    pltpu.sync_copy(dbuf, deg_sh.at[pl.ds(roff, _ROWS)])
    plsc.subcore_barrier()

    ebase = wid * EPW

    def chunk(k, carry):
        b = ebase + k * CH
        pltpu.sync_copy(dst_hbm.at[pl.ds(b, CH)], dstv)
        pltpu.sync_copy(w_hbm.at[pl.ds(b, CH)], w_v)
        pltpu.sync_copy(w_v, deg_sh.at[dstv], add=True)
        return carry

    lax.fori_loop(0, NCH, chunk, 0)
    plsc.subcore_barrier()

    pltpu.sync_copy(deg_sh.at[pl.ds(roff, _ROWS)], dbuf)
    pltpu.sync_copy(dbuf, out_hbm.at[pl.ds(cid * N + roff, _ROWS)])


_sc_deg = pl.kernel(
    _sc_deg_body,
    out_type=jax.ShapeDtypeStruct((NC * N,), jnp.float32),
    mesh=_mesh,
    scratch_types=[
        pltpu.VMEM((CH,), jnp.int32),
        pltpu.VMEM((CH,), jnp.float32),
        pltpu.VMEM((_ROWS,), jnp.float32),
        pltpu.VMEM_SHARED((N,), jnp.float32),
    ],
)


def _sc_agg_body(g_hbm, src_hbm, dst_hbm, w_hbm, out_hbm,
                 srcv, dstv, w_v, rows_v, sem, acc_sh):
    cid = lax.axis_index("c")
    sid = lax.axis_index("s")
    wid = sid * NC + cid
    roff = sid * _STRIDE

    def zrow(r, carry):
        for j in range(8):
            rows_v[r, pl.ds(j * 16, 16)] = jnp.zeros((16,), jnp.float32)
        return carry

    lax.fori_loop(0, CH, zrow, 0)
    for t in range(_ROWS // CH):
        pltpu.sync_copy(rows_v, acc_sh.at[pl.ds(roff + t * CH, CH)])
    plsc.subcore_barrier()

    ebase = wid * EPW

    def chunk(k, carry):
        b = ebase + k * CH
        pltpu.sync_copy(src_hbm.at[pl.ds(b, CH)], srcv)
        pltpu.sync_copy(dst_hbm.at[pl.ds(b, CH)], dstv)
        pltpu.sync_copy(w_hbm.at[pl.ds(b, CH)], w_v)
        pltpu.async_copy(g_hbm.at[srcv], rows_v, sem).wait()

        def egroup(gi, c2):
            w16 = w_v[pl.ds(gi * 16, 16)]
            for e in range(16):
                sp = jnp.broadcast_to(lax.slice(w16, (e,), (e + 1,)), (16,))
                r = gi * 16 + e
                for j in range(8):
                    sl = pl.ds(j * 16, 16)
                    rows_v[r, sl] = rows_v[r, sl] * sp
            return c2

        lax.fori_loop(0, CH // 16, egroup, 0)
        pltpu.sync_copy(rows_v, acc_sh.at[dstv], add=True)
        return carry

    lax.fori_loop(0, NCH, chunk, 0)
    plsc.subcore_barrier()

    for t in range(_ROWS // CH):
        pltpu.sync_copy(acc_sh.at[pl.ds(roff + t * CH, CH)], rows_v)
        pltpu.sync_copy(rows_v, out_hbm.at[cid, pl.ds(roff + t * CH, CH)])


_sc_agg = pl.kernel(
    _sc_agg_body,
    out_type=jax.ShapeDtypeStruct((NC, N, D), jnp.float32),
    mesh=_mesh,
    scratch_types=[
        pltpu.VMEM((CH,), jnp.int32),
        pltpu.VMEM((CH,), jnp.int32),
        pltpu.VMEM((CH,), jnp.float32),
        pltpu.VMEM((CH, D), jnp.float32),
        pltpu.SemaphoreType.DMA,
        pltpu.VMEM_SHARED((N, D), jnp.float32),
    ],
)


def _tc_pre_body(x_ref, w_ref, deg_ref, g_ref, dis_ref):
    h = jnp.dot(x_ref[...], w_ref[...], preferred_element_type=jnp.float32)
    deg = deg_ref[0, :, 0] + deg_ref[1, :, 0] + 1.0
    dis = lax.rsqrt(deg)
    g_ref[...] = h * dis[:, None]
    dis_ref[...] = dis[:, None]


_BR = 1000

_tc_pre = pl.pallas_call(
    _tc_pre_body,
    grid=(N // _BR,),
    in_specs=[
        pl.BlockSpec((_BR, D), lambda i: (i, 0)),
        pl.BlockSpec((D, D), lambda i: (0, 0)),
        pl.BlockSpec((NC, _BR, 1), lambda i: (0, i, 0)),
    ],
    out_specs=[
        pl.BlockSpec((_BR, D), lambda i: (i, 0)),
        pl.BlockSpec((_BR, 1), lambda i: (i, 0)),
    ],
    out_shape=[
        jax.ShapeDtypeStruct((N, D), jnp.float32),
        jax.ShapeDtypeStruct((N, 1), jnp.float32),
    ],
)


def _tc_post_body(acc_ref, g_ref, dis_ref, b_ref, o_ref):
    s = acc_ref[0] + acc_ref[1] + g_ref[...]
    o_ref[...] = jnp.maximum(s * dis_ref[...] + b_ref[...], 0.0)


_tc_post = pl.pallas_call(
    _tc_post_body,
    grid=(N // _BR,),
    in_specs=[
        pl.BlockSpec((NC, _BR, D), lambda i: (0, i, 0)),
        pl.BlockSpec((_BR, D), lambda i: (i, 0)),
        pl.BlockSpec((_BR, 1), lambda i: (i, 0)),
        pl.BlockSpec((1, D), lambda i: (0, 0)),
    ],
    out_specs=pl.BlockSpec((_BR, D), lambda i: (i, 0)),
    out_shape=jax.ShapeDtypeStruct((N, D), jnp.float32),
)


def kernel(x, edge_index, edge_weights, W, b):
    src = edge_index[0]
    dst = edge_index[1]
    deg2 = _sc_deg(dst, edge_weights)
    g, dis = _tc_pre(x, W, deg2.reshape(NC, N, 1))
    acc2 = _sc_agg(g, src, dst, edge_weights)
    return _tc_post(acc2, g, dis, b.reshape(1, D))


# speedup vs baseline: 14.4488x; 1.1057x over previous
import jax
import jax.numpy as jnp
from jax import lax
from jax.experimental import pallas as pl
from jax.experimental.pallas import tpu as pltpu
from jax.experimental.pallas import tpu_sc as plsc

N = 10000
E = 320000
D = 128

NC = 2
NS = 16
NW = NC * NS
CH = 64
NCH = 160
EPW = NCH * CH
EPAD = NW * EPW

_SHIFT = 14
_MASK = (1 << _SHIFT) - 1

_STRIDE = 624
_ROWS = 640

_mesh = plsc.VectorSubcoreMesh(core_axis_name="c", subcore_axis_name="s")


def _sc_deg_body(pk_hbm, w_hbm, out_hbm, pm, wm, dstm, dbuf, sem, deg_sh):
    cid = lax.axis_index("c")
    sid = lax.axis_index("s")
    wid = sid * NC + cid
    roff = sid * _STRIDE

    for t in range(_ROWS // 16):
        dbuf[pl.ds(t * 16, 16)] = jnp.zeros((16,), jnp.float32)
    pltpu.sync_copy(dbuf, deg_sh.at[pl.ds(roff, _ROWS)])

    pltpu.sync_copy(pk_hbm.at[pl.ds(wid * NCH, NCH)], pm)
    pltpu.sync_copy(w_hbm.at[pl.ds(wid * NCH, NCH)], wm)

    def unpack(j, carry):
        for t in range(CH // 16):
            sl = pl.ds(t * 16, 16)
            dstm[j, sl] = jnp.bitwise_and(pm[j, sl], _MASK)
        return carry

    lax.fori_loop(0, NCH, unpack, 0)
    plsc.subcore_barrier()

    def group(gg, carry):
        for t in range(8):
            j = gg * 8 + t
            pltpu.async_copy(wm.at[j], deg_sh.at[dstm.at[j]], sem, add=True)
        for t in range(8):
            j = gg * 8 + t
            pltpu.make_async_copy(wm.at[j], deg_sh.at[dstm.at[j]], sem).wait()
        return carry

    lax.fori_loop(0, NCH // 8, group, 0)
    plsc.subcore_barrier()

    pltpu.sync_copy(deg_sh.at[pl.ds(roff, _ROWS)], dbuf)
    pltpu.sync_copy(dbuf, out_hbm.at[pl.ds(cid * N + roff, _ROWS)])


_sc_deg = pl.kernel(
    _sc_deg_body,
    out_type=jax.ShapeDtypeStruct((NC * N,), jnp.float32),
    mesh=_mesh,
    scratch_types=[
        pltpu.VMEM((NCH, CH), jnp.int32),
        pltpu.VMEM((NCH, CH), jnp.float32),
        pltpu.VMEM((NCH, CH), jnp.int32),
        pltpu.VMEM((_ROWS,), jnp.float32),
        pltpu.SemaphoreType.DMA,
        pltpu.VMEM_SHARED((N,), jnp.float32),
    ],
)


def _sc_agg_body(g_hbm, pk_hbm, w_hbm, out_hbm,
                 pkv0, pkv1, wv0, wv1, srcv0, srcv1, dstv0, dstv1,
                 rows0, rows1, gsem0, gsem1, isem0, isem1, acc_sh):
    cid = lax.axis_index("c")
    sid = lax.axis_index("s")
    wid = sid * NC + cid
    roff = sid * _STRIDE
    ebase = wid * EPW

    def zrow(r, carry):
        for j in range(8):
            rows0[r, pl.ds(j * 16, 16)] = jnp.zeros((16,), jnp.float32)
        return carry

    lax.fori_loop(0, CH, zrow, 0)
    for t in range(_ROWS // CH):
        pltpu.sync_copy(rows0, acc_sh.at[pl.ds(roff + t * CH, CH)])
    plsc.subcore_barrier()

    def issue_idx(j, pkv, wv, isem):
        b = ebase + j * CH
        pltpu.async_copy(pk_hbm.at[pl.ds(b, CH)], pkv, isem)
        pltpu.async_copy(w_hbm.at[pl.ds(b, CH)], wv, isem)

    def wait_idx(pkv, wv, isem):
        pltpu.make_async_copy(pk_hbm.at[pl.ds(0, CH)], pkv, isem).wait()
        pltpu.make_async_copy(w_hbm.at[pl.ds(0, CH)], wv, isem).wait()

    def unpack(pkv, srcv, dstv):
        for t in range(CH // 16):
            sl = pl.ds(t * 16, 16)
            p16 = pkv[sl]
            dstv[sl] = jnp.bitwise_and(p16, _MASK)
            srcv[sl] = lax.shift_right_logical(p16, _SHIFT)

    def scale(wv, rows_v):
        def egroup(gi, c2):
            w16 = wv[pl.ds(gi * 16, 16)]
            for e in range(16):
                sp = jnp.broadcast_to(lax.slice(w16, (e,), (e + 1,)), (16,))
                r = gi * 16 + e
                for jj in range(8):
                    sl = pl.ds(jj * 16, 16)
                    rows_v[r, sl] = rows_v[r, sl] * sp
            return c2
        lax.fori_loop(0, CH // 16, egroup, 0)

    def wait_gather(rows_v, gsem):
        pltpu.make_async_copy(g_hbm.at[pl.ds(0, CH)], rows_v, gsem).wait()

    issue_idx(0, pkv0, wv0, isem0)
    issue_idx(1, pkv1, wv1, isem1)
    wait_idx(pkv0, wv0, isem0)
    unpack(pkv0, srcv0, dstv0)
    pltpu.async_copy(g_hbm.at[srcv0], rows0, gsem0)
    wait_idx(pkv1, wv1, isem1)
    unpack(pkv1, srcv1, dstv1)
    pltpu.async_copy(g_hbm.at[srcv1], rows1, gsem1)

    def slot(j, pkv, wv, srcv, dstv, rows_v, gsem, isem):
        wait_gather(rows_v, gsem)
        scale(wv, rows_v)

        @pl.when(j + 2 < NCH)
        def _():
            issue_idx(j + 2, pkv, wv, isem)

        pltpu.sync_copy(rows_v, acc_sh.at[dstv], add=True)

        @pl.when(j + 2 < NCH)
        def _():
            wait_idx(pkv, wv, isem)
            unpack(pkv, srcv, dstv)
            pltpu.async_copy(g_hbm.at[srcv], rows_v, gsem)

    def pair(gpair, carry):
        j0 = gpair * 2
        slot(j0, pkv0, wv0, srcv0, dstv0, rows0, gsem0, isem0)
        slot(j0 + 1, pkv1, wv1, srcv1, dstv1, rows1, gsem1, isem1)
        return carry

    lax.fori_loop(0, NCH // 2, pair, 0)
    plsc.subcore_barrier()

    for t in range(_ROWS // CH):
        pltpu.sync_copy(acc_sh.at[pl.ds(roff + t * CH, CH)], rows0)
        pltpu.sync_copy(rows0, out_hbm.at[cid, pl.ds(roff + t * CH, CH)])


_sc_agg = pl.kernel(
    _sc_agg_body,
    out_type=jax.ShapeDtypeStruct((NC, N, D), jnp.float32),
    mesh=_mesh,
    scratch_types=[
        pltpu.VMEM((CH,), jnp.int32),
        pltpu.VMEM((CH,), jnp.int32),
        pltpu.VMEM((CH,), jnp.float32),
        pltpu.VMEM((CH,), jnp.float32),
        pltpu.VMEM((CH,), jnp.int32),
        pltpu.VMEM((CH,), jnp.int32),
        pltpu.VMEM((CH,), jnp.int32),
        pltpu.VMEM((CH,), jnp.int32),
        pltpu.VMEM((CH, D), jnp.float32),
        pltpu.VMEM((CH, D), jnp.float32),
        pltpu.SemaphoreType.DMA,
        pltpu.SemaphoreType.DMA,
        pltpu.SemaphoreType.DMA,
        pltpu.SemaphoreType.DMA,
        pltpu.VMEM_SHARED((N, D), jnp.float32),
    ],
)


def _tc_pre_body(x_ref, w_ref, deg_ref, g_ref, dis_ref):
    h = jnp.dot(x_ref[...], w_ref[...], preferred_element_type=jnp.float32)
    deg = deg_ref[0, :, 0] + deg_ref[1, :, 0] + 1.0
    dis = lax.rsqrt(deg)
    g_ref[...] = h * dis[:, None]
    dis_ref[...] = dis[:, None]


_BR = 1000

_tc_pre = pl.pallas_call(
    _tc_pre_body,
    grid=(N // _BR,),
    in_specs=[
        pl.BlockSpec((_BR, D), lambda i: (i, 0)),
        pl.BlockSpec((D, D), lambda i: (0, 0)),
        pl.BlockSpec((NC, _BR, 1), lambda i: (0, i, 0)),
    ],
    out_specs=[
        pl.BlockSpec((_BR, D), lambda i: (i, 0)),
        pl.BlockSpec((_BR, 1), lambda i: (i, 0)),
    ],
    out_shape=[
        jax.ShapeDtypeStruct((N, D), jnp.float32),
        jax.ShapeDtypeStruct((N, 1), jnp.float32),
    ],
)


def _tc_post_body(acc_ref, g_ref, dis_ref, b_ref, o_ref):
    s = acc_ref[0] + acc_ref[1] + g_ref[...]
    o_ref[...] = jnp.maximum(s * dis_ref[...] + b_ref[...], 0.0)


_tc_post = pl.pallas_call(
    _tc_post_body,
    grid=(N // _BR,),
    in_specs=[
        pl.BlockSpec((NC, _BR, D), lambda i: (0, i, 0)),
        pl.BlockSpec((_BR, D), lambda i: (i, 0)),
        pl.BlockSpec((_BR, 1), lambda i: (i, 0)),
        pl.BlockSpec((1, D), lambda i: (0, 0)),
    ],
    out_specs=pl.BlockSpec((_BR, D), lambda i: (i, 0)),
    out_shape=jax.ShapeDtypeStruct((N, D), jnp.float32),
)


def kernel(x, edge_index, edge_weights, W, b):
    src = edge_index[0]
    dst = edge_index[1]
    packed = src * (1 << _SHIFT) + dst
    pad2 = lambda a: jnp.pad(a, (0, EPAD - E)).reshape(NW * NCH, CH)
    pk2 = pad2(packed)
    w2 = pad2(edge_weights)
    deg2 = _sc_deg(pk2, w2)
    g, dis = _tc_pre(x, W, deg2.reshape(NC, N, 1))
    acc2 = _sc_agg(g, pk2.reshape(-1), w2.reshape(-1))
    return _tc_post(acc2, g, dis, b.reshape(1, D))
